# baseline (speedup 1.0000x reference)
import functools

import jax
import jax.numpy as jnp
from jax import lax
from jax.experimental import pallas as pl
from jax.experimental.pallas import tpu as pltpu
from jax.experimental.pallas import tpu_sc as plsc

NE = 8
NB = 2
NS = 2048
ND = 1024
NH = 4096
NL = NS // 2
EPSG = 1e-06
NW = 32
RPW = (NE * NB * NL) // NW
GCH = 64
NCH = RPW // GCH
DH = ND // 2
NPB = NL // 256
DUMMY = NE * NB * NL
NMR = DUMMY + NPB * 256
NGR = NE * NB * NS
GRPW = NGR // NW
GRCH = GRPW // 128


def _routing_body(xb_ref, wg_ref, bg_ref, route_ref):
    logits = jnp.dot(xb_ref[...], wg_ref[...], preferred_element_type=jnp.float32)
    logits = logits + bg_ref[...]
    m = jnp.max(logits, axis=1, keepdims=True)
    ex = jnp.exp(logits - m)
    gate = ex / jnp.sum(ex, axis=1, keepdims=True)
    e_iota = jax.lax.broadcasted_iota(jnp.int32, (NB * NS, NE), 1)
    cnt = jnp.zeros((NB * NS, NE), dtype=jnp.float32)
    for f in range(NE):
        lf = logits[:, f : f + 1]
        beats = (lf > logits) | ((lf == logits) & (f < e_iota))
        cnt = cnt + beats.astype(jnp.float32)
    mask = (cnt < 2.0).astype(jnp.float32)
    masked = gate * mask
    m0 = masked[:NS, :]
    m1 = masked[NS:, :]
    denom = m0 + m1 + EPSG
    route_ref[:NS, :] = m0 / denom * 2.0
    route_ref[NS:, :] = m1 / denom * 2.0


def _select_body(rrow_ref, rcol_ref, gid_ref, ctb_ref):
    p = pl.program_id(0)
    b = p % NB
    rrow = rrow_ref[0]
    rcol = rcol_ref[0]
    s_iota = jax.lax.broadcasted_iota(jnp.int32, (256, NS), 1)
    rank = jnp.zeros((1, NS), dtype=jnp.float32)
    for tb in range(NS // 256):
        tcol = rcol[tb * 256 : (tb + 1) * 256, :]
        t_iota = jax.lax.broadcasted_iota(jnp.int32, (256, NS), 0) + tb * 256
        beats = (tcol > rrow) | ((tcol == rrow) & (t_iota < s_iota))
        rank = rank + jnp.sum(beats.astype(jnp.float32), axis=0, keepdims=True)
    sel_row = (rank < float(NL)).astype(jnp.float32)
    sel_col = jnp.transpose(sel_row)
    c_row = jnp.zeros((1, NS), dtype=jnp.float32)
    for tb in range(NS // 256):
        selc = sel_col[tb * 256 : (tb + 1) * 256, :]
        t_iota = jax.lax.broadcasted_iota(jnp.int32, (256, NS), 0) + tb * 256
        le = (t_iota <= s_iota).astype(jnp.float32)
        c_row = c_row + jnp.sum(selc * le, axis=0, keepdims=True)
    ctb_ref[0] = jnp.where(
        sel_row > 0.0,
        p * NL + c_row.astype(jnp.int32) - 1,
        DUMMY,
    )
    c_col = jnp.transpose(c_row)
    seq = jnp.zeros((1, NL), dtype=jnp.float32)
    l_iota = jax.lax.broadcasted_iota(jnp.int32, (256, NL), 1).astype(jnp.float32)
    for sb in range(NS // 256):
        c_blk = c_col[sb * 256 : (sb + 1) * 256, :]
        seq = seq + jnp.sum((c_blk <= l_iota).astype(jnp.float32), axis=0, keepdims=True)
    gid_ref[0] = seq.astype(jnp.int32) + b * NS


def _sc_gather_body(xf_hbm, gid_hbm, xg_hbm, idx_v, rows_v):
    wid = lax.axis_index("s") * 2 + lax.axis_index("c")
    pltpu.sync_copy(gid_hbm.at[wid], idx_v)
    for j in range(NCH):
        pltpu.sync_copy(xf_hbm.at[idx_v.at[j]], rows_v)
        pltpu.sync_copy(rows_v, xg_hbm.at[pl.ds(wid * RPW + j * GCH, GCH)])


def _sc_combine_body(mlpa_hbm, mlpb_hbm, ctb_hbm, outa_hbm, outb_hbm,
                     idx_v, buf_v):
    wid = lax.axis_index("s") * 2 + lax.axis_index("c")
    pltpu.sync_copy(ctb_hbm.at[wid], idx_v)
    for mlp_hbm, outs_hbm in ((mlpa_hbm, outa_hbm), (mlpb_hbm, outb_hbm)):
        for j in range(GRCH):
            pltpu.sync_copy(mlp_hbm.at[idx_v.at[j]], buf_v)
            pltpu.sync_copy(buf_v, outs_hbm.at[pl.ds(wid * GRPW + j * 128, 128)])


@functools.cache
def _sc_kernels():
    mesh = plsc.VectorSubcoreMesh(core_axis_name="c", subcore_axis_name="s")
    gather = functools.partial(
        pl.kernel,
        mesh=mesh,
        out_type=jax.ShapeDtypeStruct((NE * NB * NL, ND), jnp.float32),
        scratch_types=[
            pltpu.VMEM((NCH, GCH), jnp.int32),
            pltpu.VMEM((GCH, ND), jnp.float32),
        ],
    )(_sc_gather_body)
    combine = functools.partial(
        pl.kernel,
        mesh=mesh,
        out_type=[
            jax.ShapeDtypeStruct((NGR, DH), jnp.float32),
            jax.ShapeDtypeStruct((NGR, DH), jnp.float32),
        ],
        scratch_types=[
            pltpu.VMEM((GRCH, 128), jnp.int32),
            pltpu.VMEM((128, DH), jnp.float32),
        ],
    )(_sc_combine_body)
    return gather, combine


def _sc_gather(xf, gid_w):
    return _sc_kernels()[0](xf, gid_w)


def _sc_combine(mlpa, mlpb, ctb_w):
    return _sc_kernels()[1](mlpa, mlpb, ctb_w)


def _mlp_body(xg_ref, w1_ref, w2_ref, b1_ref, b2_ref, outa_ref, outb_ref):
    p = pl.program_id(0)

    @pl.when(p >= NE * NB)
    def _():
        outa_ref[...] = jnp.zeros_like(outa_ref)
        outb_ref[...] = jnp.zeros_like(outb_ref)

    @pl.when(p < NE * NB)
    def _():
        xb = xg_ref[...].astype(jnp.bfloat16)
        h = jnp.dot(xb, w1_ref[0], preferred_element_type=jnp.float32)
        h = h + b1_ref[0, 0]
        h = 0.5 * h * (1.0 + jax.lax.erf(h * 0.7071067811865476))
        hb = h.astype(jnp.bfloat16)
        outa_ref[...] = (
            jnp.dot(hb, w2_ref[0, :, :DH], preferred_element_type=jnp.float32)
            + b2_ref[0, 0, :DH])
        outb_ref[...] = (
            jnp.dot(hb, w2_ref[0, :, DH:], preferred_element_type=jnp.float32)
            + b2_ref[0, 0, DH:])


def _ln_body(oa_ref, ob_ref, x_ref, g_ref, b_ref, out_ref):
    ga = oa_ref[...]
    gb = ob_ref[...]
    za = x_ref[:, :DH]
    zb = x_ref[:, DH:]
    for r in range(NE):
        za = za + ga[r]
        zb = zb + gb[r]
    mean = (jnp.sum(za, axis=-1, keepdims=True) + jnp.sum(zb, axis=-1, keepdims=True)) / ND
    za = za - mean
    zb = zb - mean
    var = (jnp.sum(za * za, axis=-1, keepdims=True) + jnp.sum(zb * zb, axis=-1, keepdims=True)) / ND
    rs = jax.lax.rsqrt(var + 1e-05)
    out_ref[:, :DH] = za * rs * g_ref[:, :DH] + b_ref[:, :DH]
    out_ref[:, DH:] = zb * rs * g_ref[:, DH:] + b_ref[:, DH:]


@functools.partial(jax.jit, static_argnums=())
def kernel(x, w_gate, b_gate, W1, b1, W2, b2, ln_gamma, ln_beta):
    B, S, D = x.shape
    H = W1.shape[2]
    xf = x.reshape(B * S, D)
    xb16 = xf.astype(jnp.bfloat16)

    route = pl.pallas_call(
        _routing_body,
        out_shape=jax.ShapeDtypeStruct((B * S, NE), jnp.float32),
    )(xb16, w_gate.astype(jnp.bfloat16), b_gate.reshape(1, NE))

    route_eb = jnp.transpose(route).reshape(NE * B, S)
    rrow = route_eb.reshape(NE * B, 1, S)
    rcol = route_eb.reshape(NE * B, S, 1)

    gid, ctb = pl.pallas_call(
        _select_body,
        out_shape=[
            jax.ShapeDtypeStruct((NE * B, 1, NL), jnp.int32),
            jax.ShapeDtypeStruct((NE * B, 1, S), jnp.int32),
        ],
        grid=(NE * B,),
        in_specs=[
            pl.BlockSpec((1, 1, S), lambda i: (i, 0, 0)),
            pl.BlockSpec((1, S, 1), lambda i: (i, 0, 0)),
        ],
        out_specs=[
            pl.BlockSpec((1, 1, NL), lambda i: (i, 0, 0)),
            pl.BlockSpec((1, 1, S), lambda i: (i, 0, 0)),
        ],
    )(rrow, rcol)

    gid_w = gid.reshape(NW, NCH, GCH)
    ctb_w = ctb.reshape(NW, GRCH, 128)

    xg = _sc_gather(xf, gid_w)

    mlpa, mlpb = pl.pallas_call(
        _mlp_body,
        out_shape=[
            jax.ShapeDtypeStruct((NMR, DH), jnp.float32),
            jax.ShapeDtypeStruct((NMR, DH), jnp.float32),
        ],
        grid=(NE * B + 1, NPB),
        in_specs=[
            pl.BlockSpec((256, D), lambda p, r: ((p - p // (NE * NB)) * NPB + r, 0)),
            pl.BlockSpec((1, D, H), lambda p, r: ((p - p // (NE * NB)) // 2, 0, 0)),
            pl.BlockSpec((1, H, D), lambda p, r: ((p - p // (NE * NB)) // 2, 0, 0)),
            pl.BlockSpec((1, 1, H), lambda p, r: ((p - p // (NE * NB)) // 2, 0, 0)),
            pl.BlockSpec((1, 1, D), lambda p, r: ((p - p // (NE * NB)) // 2, 0, 0)),
        ],
        out_specs=[
            pl.BlockSpec((256, DH), lambda p, r: (p * NPB + r, 0)),
            pl.BlockSpec((256, DH), lambda p, r: (p * NPB + r, 0)),
        ],
    )(
        xg,
        W1.astype(jnp.bfloat16),
        W2.astype(jnp.bfloat16),
        b1.reshape(NE, 1, H),
        b2.reshape(NE, 1, D),
    )

    garra, garrb = _sc_combine(mlpa, mlpb, ctb_w)

    TB = (B * S) // 16
    out_ln = pl.pallas_call(
        _ln_body,
        out_shape=jax.ShapeDtypeStruct((B * S, D), jnp.float32),
        grid=(16,),
        in_specs=[
            pl.BlockSpec((NE, TB, DH), lambda i: (0, i, 0)),
            pl.BlockSpec((NE, TB, DH), lambda i: (0, i, 0)),
            pl.BlockSpec((TB, D), lambda i: (i, 0)),
            pl.BlockSpec((1, D), lambda i: (0, 0)),
            pl.BlockSpec((1, D), lambda i: (0, 0)),
        ],
        out_specs=pl.BlockSpec((TB, D), lambda i: (i, 0)),
    )(garra.reshape(NE, B * S, DH), garrb.reshape(NE, B * S, DH),
      xf, ln_gamma.reshape(1, D), ln_beta.reshape(1, D))
    return out_ln.reshape(B, S, D)


# speedup vs baseline: 2.4305x; 2.4305x over previous
import functools

import jax
import jax.numpy as jnp
from jax import lax
from jax.experimental import pallas as pl
from jax.experimental.pallas import tpu as pltpu
from jax.experimental.pallas import tpu_sc as plsc

NE = 8
NB = 2
NS = 2048
ND = 1024
NH = 4096
NL = NS // 2
EPSG = 1e-06
NW = 32
RPW = (NE * NB * NL) // NW
GCH = 64
NCH = RPW // GCH
DH = ND // 2
NPB = NL // 256
DUMMY = NE * NB * NL
NMR = DUMMY + NPB * 256
NGR = NE * NB * NS
GRPW = NGR // NW
GRCH = GRPW // GCH


def _routing_body(xb_ref, wg_ref, bg_ref, route_ref):
    logits = jnp.dot(xb_ref[...], wg_ref[...], preferred_element_type=jnp.float32)
    logits = logits + bg_ref[...]
    m = jnp.max(logits, axis=1, keepdims=True)
    ex = jnp.exp(logits - m)
    gate = ex / jnp.sum(ex, axis=1, keepdims=True)
    e_iota = jax.lax.broadcasted_iota(jnp.int32, (NB * NS, NE), 1)
    cnt = jnp.zeros((NB * NS, NE), dtype=jnp.float32)
    for f in range(NE):
        lf = logits[:, f : f + 1]
        beats = (lf > logits) | ((lf == logits) & (f < e_iota))
        cnt = cnt + beats.astype(jnp.float32)
    mask = (cnt < 2.0).astype(jnp.float32)
    masked = gate * mask
    m0 = masked[:NS, :]
    m1 = masked[NS:, :]
    denom = m0 + m1 + EPSG
    route_ref[:NS, :] = m0 / denom * 2.0
    route_ref[NS:, :] = m1 / denom * 2.0


def _select_body(rrow_ref, rcol_ref, gid_ref, ctb_ref):
    p = pl.program_id(0)
    b = p % NB
    rrow = rrow_ref[0]
    rcol = rcol_ref[0]
    s_iota = jax.lax.broadcasted_iota(jnp.int32, (256, NS), 1)
    rank = jnp.zeros((1, NS), dtype=jnp.float32)
    for tb in range(NS // 256):
        tcol = rcol[tb * 256 : (tb + 1) * 256, :]
        t_iota = jax.lax.broadcasted_iota(jnp.int32, (256, NS), 0) + tb * 256
        beats = (tcol > rrow) | ((tcol == rrow) & (t_iota < s_iota))
        rank = rank + jnp.sum(beats.astype(jnp.float32), axis=0, keepdims=True)
    sel_row = (rank < float(NL)).astype(jnp.float32)
    sel_col = jnp.transpose(sel_row)
    c_row = jnp.zeros((1, NS), dtype=jnp.float32)
    for tb in range(NS // 256):
        selc = sel_col[tb * 256 : (tb + 1) * 256, :]
        t_iota = jax.lax.broadcasted_iota(jnp.int32, (256, NS), 0) + tb * 256
        le = (t_iota <= s_iota).astype(jnp.float32)
        c_row = c_row + jnp.sum(selc * le, axis=0, keepdims=True)
    lane = jax.lax.broadcasted_iota(jnp.int32, (1, NS), 1)
    ctb_ref[0] = jnp.where(
        sel_row > 0.0,
        p * NL + c_row.astype(jnp.int32) - 1,
        DUMMY + (lane % NL),
    )
    c_col = jnp.transpose(c_row)
    seq = jnp.zeros((1, NL), dtype=jnp.float32)
    l_iota = jax.lax.broadcasted_iota(jnp.int32, (256, NL), 1).astype(jnp.float32)
    for sb in range(NS // 256):
        c_blk = c_col[sb * 256 : (sb + 1) * 256, :]
        seq = seq + jnp.sum((c_blk <= l_iota).astype(jnp.float32), axis=0, keepdims=True)
    gid_ref[0] = seq.astype(jnp.int32) + b * NS


def _sc_gather_body(xf_hbm, gid_hbm, xg_hbm, idx_v, rows_v):
    wid = lax.axis_index("s") * 2 + lax.axis_index("c")
    pltpu.sync_copy(gid_hbm.at[wid], idx_v)
    for j in range(NCH):
        pltpu.sync_copy(xf_hbm.at[idx_v.at[j]], rows_v)
        pltpu.sync_copy(rows_v, xg_hbm.at[pl.ds(wid * RPW + j * GCH, GCH)])


def _sc_combine_body(mlp_hbm, ctb_hbm, garr_hbm, idx_v, buf_v):
    wid = lax.axis_index("s") * 2 + lax.axis_index("c")
    pltpu.sync_copy(ctb_hbm.at[wid], idx_v)
    for j in range(GRCH):
        pltpu.sync_copy(mlp_hbm.at[idx_v.at[j]], buf_v)
        pltpu.sync_copy(buf_v, garr_hbm.at[pl.ds(wid * GRPW + j * GCH, GCH)])


@functools.cache
def _sc_kernels():
    mesh = plsc.VectorSubcoreMesh(core_axis_name="c", subcore_axis_name="s")
    gather = functools.partial(
        pl.kernel,
        mesh=mesh,
        out_type=jax.ShapeDtypeStruct((NE * NB * NL, ND), jnp.float32),
        scratch_types=[
            pltpu.VMEM((NCH, GCH), jnp.int32),
            pltpu.VMEM((GCH, ND), jnp.float32),
        ],
    )(_sc_gather_body)
    combine = functools.partial(
        pl.kernel,
        mesh=mesh,
        out_type=jax.ShapeDtypeStruct((NGR, ND), jnp.float32),
        scratch_types=[
            pltpu.VMEM((GRCH, GCH), jnp.int32),
            pltpu.VMEM((GCH, ND), jnp.float32),
        ],
    )(_sc_combine_body)
    return gather, combine


def _sc_gather(xf, gid_w):
    return _sc_kernels()[0](xf, gid_w)


def _sc_combine(mlp, ctb_w):
    return _sc_kernels()[1](mlp, ctb_w)


def _mlp_body(xg_ref, w1_ref, w2_ref, b1_ref, b2_ref, out_ref):
    p = pl.program_id(0)

    @pl.when(p >= NE * NB)
    def _():
        out_ref[...] = jnp.zeros_like(out_ref)

    @pl.when(p < NE * NB)
    def _():
        xb = xg_ref[...].astype(jnp.bfloat16)
        h = jnp.dot(xb, w1_ref[0], preferred_element_type=jnp.float32)
        h = h + b1_ref[0, 0]
        h = 0.5 * h * (1.0 + jax.lax.erf(h * 0.7071067811865476))
        hb = h.astype(jnp.bfloat16)
        out_ref[...] = (
            jnp.dot(hb, w2_ref[0], preferred_element_type=jnp.float32)
            + b2_ref[0, 0])


def _ln_body(ga_ref, x_ref, g_ref, b_ref, out_ref):
    z = x_ref[...]
    for r in range(NE):
        z = z + ga_ref[r]
    mean = jnp.mean(z, axis=-1, keepdims=True)
    z = z - mean
    var = jnp.mean(z * z, axis=-1, keepdims=True)
    out_ref[...] = z * jax.lax.rsqrt(var + 1e-05) * g_ref[...] + b_ref[...]


@functools.partial(jax.jit, static_argnums=())
def kernel(x, w_gate, b_gate, W1, b1, W2, b2, ln_gamma, ln_beta):
    B, S, D = x.shape
    H = W1.shape[2]
    xf = x.reshape(B * S, D)
    xb16 = xf.astype(jnp.bfloat16)

    route = pl.pallas_call(
        _routing_body,
        out_shape=jax.ShapeDtypeStruct((B * S, NE), jnp.float32),
    )(xb16, w_gate.astype(jnp.bfloat16), b_gate.reshape(1, NE))

    route_eb = jnp.transpose(route).reshape(NE * B, S)
    rrow = route_eb.reshape(NE * B, 1, S)
    rcol = route_eb.reshape(NE * B, S, 1)

    gid, ctb = pl.pallas_call(
        _select_body,
        out_shape=[
            jax.ShapeDtypeStruct((NE * B, 1, NL), jnp.int32),
            jax.ShapeDtypeStruct((NE * B, 1, S), jnp.int32),
        ],
        grid=(NE * B,),
        in_specs=[
            pl.BlockSpec((1, 1, S), lambda i: (i, 0, 0)),
            pl.BlockSpec((1, S, 1), lambda i: (i, 0, 0)),
        ],
        out_specs=[
            pl.BlockSpec((1, 1, NL), lambda i: (i, 0, 0)),
            pl.BlockSpec((1, 1, S), lambda i: (i, 0, 0)),
        ],
    )(rrow, rcol)

    gid_w = gid.reshape(NW, NCH, GCH)
    ctb_w = ctb.reshape(NW, GRCH, GCH)

    xg = _sc_gather(xf, gid_w)

    mlp = pl.pallas_call(
        _mlp_body,
        out_shape=jax.ShapeDtypeStruct((NMR, D), jnp.float32),
        grid=(NE * B + 1, NPB),
        in_specs=[
            pl.BlockSpec((256, D), lambda p, r: ((p - p // (NE * NB)) * NPB + r, 0)),
            pl.BlockSpec((1, D, H), lambda p, r: ((p - p // (NE * NB)) // 2, 0, 0)),
            pl.BlockSpec((1, H, D), lambda p, r: ((p - p // (NE * NB)) // 2, 0, 0)),
            pl.BlockSpec((1, 1, H), lambda p, r: ((p - p // (NE * NB)) // 2, 0, 0)),
            pl.BlockSpec((1, 1, D), lambda p, r: ((p - p // (NE * NB)) // 2, 0, 0)),
        ],
        out_specs=pl.BlockSpec((256, D), lambda p, r: (p * NPB + r, 0)),
    )(
        xg,
        W1.astype(jnp.bfloat16),
        W2.astype(jnp.bfloat16),
        b1.reshape(NE, 1, H),
        b2.reshape(NE, 1, D),
    )

    garr = _sc_combine(mlp, ctb_w)

    TB = (B * S) // 32
    out_ln = pl.pallas_call(
        _ln_body,
        out_shape=jax.ShapeDtypeStruct((B * S, D), jnp.float32),
        grid=(32,),
        in_specs=[
            pl.BlockSpec((NE, TB, D), lambda i: (0, i, 0)),
            pl.BlockSpec((TB, D), lambda i: (i, 0)),
            pl.BlockSpec((1, D), lambda i: (0, 0)),
            pl.BlockSpec((1, D), lambda i: (0, 0)),
        ],
        out_specs=pl.BlockSpec((TB, D), lambda i: (i, 0)),
    )(garr.reshape(NE, B * S, D), xf,
      ln_gamma.reshape(1, D), ln_beta.reshape(1, D))
    return out_ln.reshape(B, S, D)


# speedup vs baseline: 2.5543x; 1.0509x over previous
import functools

import jax
import jax.numpy as jnp
from jax import lax
from jax.experimental import pallas as pl
from jax.experimental.pallas import tpu as pltpu
from jax.experimental.pallas import tpu_sc as plsc

NE = 8
NB = 2
NS = 2048
ND = 1024
NH = 4096
NL = NS // 2
EPSG = 1e-06
NW = 32
RPW = (NE * NB * NL) // NW
GCH = 64
NCH = RPW // GCH
DH = ND // 2
NPB = NL // 512
DUMMY = NE * NB * NL
NMR = DUMMY + NPB * 512
NGR = NE * NB * NS
GRPW = NGR // NW
GRCH = GRPW // GCH


def _routing_body(xb_ref, wg_ref, bg_ref, route_ref):
    logits = jnp.dot(xb_ref[...], wg_ref[...], preferred_element_type=jnp.float32)
    logits = logits + bg_ref[...]
    m = jnp.max(logits, axis=1, keepdims=True)
    ex = jnp.exp(logits - m)
    gate = ex / jnp.sum(ex, axis=1, keepdims=True)
    e_iota = jax.lax.broadcasted_iota(jnp.int32, (NB * NS, NE), 1)
    cnt = jnp.zeros((NB * NS, NE), dtype=jnp.float32)
    for f in range(NE):
        lf = logits[:, f : f + 1]
        beats = (lf > logits) | ((lf == logits) & (f < e_iota))
        cnt = cnt + beats.astype(jnp.float32)
    mask = (cnt < 2.0).astype(jnp.float32)
    masked = gate * mask
    m0 = masked[:NS, :]
    m1 = masked[NS:, :]
    denom = m0 + m1 + EPSG
    route_ref[:NS, :] = m0 / denom * 2.0
    route_ref[NS:, :] = m1 / denom * 2.0


def _select_body(rrow_ref, rcol_ref, gid_ref, ctb_ref):
    p = pl.program_id(0)
    b = p % NB
    rrow = rrow_ref[0]
    rcol = rcol_ref[0]
    s_iota = jax.lax.broadcasted_iota(jnp.int32, (256, NS), 1)
    rank = jnp.zeros((1, NS), dtype=jnp.float32)
    for tb in range(NS // 256):
        tcol = rcol[tb * 256 : (tb + 1) * 256, :]
        t_iota = jax.lax.broadcasted_iota(jnp.int32, (256, NS), 0) + tb * 256
        beats = (tcol > rrow) | ((tcol == rrow) & (t_iota < s_iota))
        rank = rank + jnp.sum(beats.astype(jnp.float32), axis=0, keepdims=True)
    sel_row = (rank < float(NL)).astype(jnp.float32)
    sel_col = jnp.transpose(sel_row)
    c_row = jnp.zeros((1, NS), dtype=jnp.float32)
    for tb in range(NS // 256):
        selc = sel_col[tb * 256 : (tb + 1) * 256, :]
        t_iota = jax.lax.broadcasted_iota(jnp.int32, (256, NS), 0) + tb * 256
        le = (t_iota <= s_iota).astype(jnp.float32)
        c_row = c_row + jnp.sum(selc * le, axis=0, keepdims=True)
    lane = jax.lax.broadcasted_iota(jnp.int32, (1, NS), 1)
    ctb_ref[0] = jnp.where(
        sel_row > 0.0,
        p * NL + c_row.astype(jnp.int32) - 1,
        DUMMY + (lane % NL),
    )
    c_col = jnp.transpose(c_row)
    seq = jnp.zeros((1, NL), dtype=jnp.float32)
    l_iota = jax.lax.broadcasted_iota(jnp.int32, (256, NL), 1).astype(jnp.float32)
    for sb in range(NS // 256):
        c_blk = c_col[sb * 256 : (sb + 1) * 256, :]
        seq = seq + jnp.sum((c_blk <= l_iota).astype(jnp.float32), axis=0, keepdims=True)
    gid_ref[0] = seq.astype(jnp.int32) + b * NS


def _sc_gather_body(xf_hbm, gid_hbm, xg_hbm, idx_v, rows_v):
    wid = lax.axis_index("s") * 2 + lax.axis_index("c")
    pltpu.sync_copy(gid_hbm.at[wid], idx_v)
    for j in range(NCH):
        pltpu.sync_copy(xf_hbm.at[idx_v.at[j]], rows_v)
        pltpu.sync_copy(rows_v, xg_hbm.at[pl.ds(wid * RPW + j * GCH, GCH)])


def _sc_combine_body(mlp_hbm, ctb_hbm, garr_hbm, idx_v, buf_v):
    wid = lax.axis_index("s") * 2 + lax.axis_index("c")
    pltpu.sync_copy(ctb_hbm.at[wid], idx_v)
    for j in range(GRCH):
        pltpu.sync_copy(mlp_hbm.at[idx_v.at[j]], buf_v)
        pltpu.sync_copy(buf_v, garr_hbm.at[pl.ds(wid * GRPW + j * GCH, GCH)])


@functools.cache
def _sc_kernels():
    mesh = plsc.VectorSubcoreMesh(core_axis_name="c", subcore_axis_name="s")
    gather = functools.partial(
        pl.kernel,
        mesh=mesh,
        out_type=jax.ShapeDtypeStruct((NE * NB * NL, ND), jnp.float32),
        scratch_types=[
            pltpu.VMEM((NCH, GCH), jnp.int32),
            pltpu.VMEM((GCH, ND), jnp.float32),
        ],
    )(_sc_gather_body)
    combine = functools.partial(
        pl.kernel,
        mesh=mesh,
        out_type=jax.ShapeDtypeStruct((NGR, ND), jnp.float32),
        scratch_types=[
            pltpu.VMEM((GRCH, GCH), jnp.int32),
            pltpu.VMEM((GCH, ND), jnp.float32),
        ],
    )(_sc_combine_body)
    return gather, combine


def _sc_gather(xf, gid_w):
    return _sc_kernels()[0](xf, gid_w)


def _sc_combine(mlp, ctb_w):
    return _sc_kernels()[1](mlp, ctb_w)


def _mlp_body(xg_ref, w1_ref, w2_ref, b1_ref, b2_ref, out_ref):
    p = pl.program_id(0)

    @pl.when(p >= NE * NB)
    def _():
        out_ref[...] = jnp.zeros_like(out_ref)

    @pl.when(p < NE * NB)
    def _():
        xb = xg_ref[...].astype(jnp.bfloat16)
        h = jnp.dot(xb, w1_ref[0], preferred_element_type=jnp.float32)
        h = h + b1_ref[0, 0]
        h = 0.5 * h * (1.0 + jax.lax.erf(h * 0.7071067811865476))
        hb = h.astype(jnp.bfloat16)
        out_ref[...] = (
            jnp.dot(hb, w2_ref[0], preferred_element_type=jnp.float32)
            + b2_ref[0, 0])


def _ln_body(ga_ref, x_ref, g_ref, b_ref, out_ref):
    z = x_ref[...]
    for r in range(NE):
        z = z + ga_ref[r]
    mean = jnp.mean(z, axis=-1, keepdims=True)
    z = z - mean
    var = jnp.mean(z * z, axis=-1, keepdims=True)
    out_ref[...] = z * jax.lax.rsqrt(var + 1e-05) * g_ref[...] + b_ref[...]


@functools.partial(jax.jit, static_argnums=())
def kernel(x, w_gate, b_gate, W1, b1, W2, b2, ln_gamma, ln_beta):
    B, S, D = x.shape
    H = W1.shape[2]
    xf = x.reshape(B * S, D)
    xb16 = xf.astype(jnp.bfloat16)

    route = pl.pallas_call(
        _routing_body,
        out_shape=jax.ShapeDtypeStruct((B * S, NE), jnp.float32),
    )(xb16, w_gate.astype(jnp.bfloat16), b_gate.reshape(1, NE))

    route_eb = jnp.transpose(route).reshape(NE * B, S)
    rrow = route_eb.reshape(NE * B, 1, S)
    rcol = route_eb.reshape(NE * B, S, 1)

    gid, ctb = pl.pallas_call(
        _select_body,
        out_shape=[
            jax.ShapeDtypeStruct((NE * B, 1, NL), jnp.int32),
            jax.ShapeDtypeStruct((NE * B, 1, S), jnp.int32),
        ],
        grid=(NE * B,),
        in_specs=[
            pl.BlockSpec((1, 1, S), lambda i: (i, 0, 0)),
            pl.BlockSpec((1, S, 1), lambda i: (i, 0, 0)),
        ],
        out_specs=[
            pl.BlockSpec((1, 1, NL), lambda i: (i, 0, 0)),
            pl.BlockSpec((1, 1, S), lambda i: (i, 0, 0)),
        ],
    )(rrow, rcol)

    gid_w = gid.reshape(NW, NCH, GCH)
    ctb_w = ctb.reshape(NW, GRCH, GCH)

    xg = _sc_gather(xf, gid_w)

    mlp = pl.pallas_call(
        _mlp_body,
        out_shape=jax.ShapeDtypeStruct((NMR, D), jnp.float32),
        grid=(NE * B + 1, NPB),
        in_specs=[
            pl.BlockSpec((512, D), lambda p, r: ((p - p // (NE * NB)) * NPB + r, 0)),
            pl.BlockSpec((1, D, H), lambda p, r: ((p - p // (NE * NB)) // 2, 0, 0)),
            pl.BlockSpec((1, H, D), lambda p, r: ((p - p // (NE * NB)) // 2, 0, 0)),
            pl.BlockSpec((1, 1, H), lambda p, r: ((p - p // (NE * NB)) // 2, 0, 0)),
            pl.BlockSpec((1, 1, D), lambda p, r: ((p - p // (NE * NB)) // 2, 0, 0)),
        ],
        out_specs=pl.BlockSpec((512, D), lambda p, r: (p * NPB + r, 0)),
    )(
        xg,
        W1.astype(jnp.bfloat16),
        W2.astype(jnp.bfloat16),
        b1.reshape(NE, 1, H),
        b2.reshape(NE, 1, D),
    )

    garr = _sc_combine(mlp, ctb_w)

    TB = (B * S) // 32
    out_ln = pl.pallas_call(
        _ln_body,
        out_shape=jax.ShapeDtypeStruct((B * S, D), jnp.float32),
        grid=(32,),
        in_specs=[
            pl.BlockSpec((NE, TB, D), lambda i: (0, i, 0)),
            pl.BlockSpec((TB, D), lambda i: (i, 0)),
            pl.BlockSpec((1, D), lambda i: (0, 0)),
            pl.BlockSpec((1, D), lambda i: (0, 0)),
        ],
        out_specs=pl.BlockSpec((TB, D), lambda i: (i, 0)),
    )(garr.reshape(NE, B * S, D), xf,
      ln_gamma.reshape(1, D), ln_beta.reshape(1, D))
    return out_ln.reshape(B, S, D)
